# baseline (speedup 1.0000x reference)
import functools

import jax
import jax.numpy as jnp
from jax import lax
from jax.experimental import pallas as pl
from jax.experimental.pallas import tpu as pltpu
from jax.experimental.pallas import tpu_sc as plsc

_NUMSTEPS = 10


def _chunk_plan(b_per_w):
    for c in range(1, b_per_w + 1):
        if b_per_w % c == 0:
            k = b_per_w // c
            if k <= 128 and k % 8 == 0:
                return c, k
    raise ValueError(f"cannot chunk {b_per_w} rows per worker")


def _sc_gather(table, idx):
    v, d = table.shape
    (n,) = idx.shape
    info = plsc.get_sparse_core_info()
    nc, ns = info.num_cores, info.num_subcores
    nw = nc * ns
    assert n % nw == 0
    b_per_w = n // nw
    c_chunks, k_chunk = _chunk_plan(b_per_w)
    idx3 = idx.reshape(nw, c_chunks, k_chunk)
    mesh = plsc.VectorSubcoreMesh(core_axis_name="c", subcore_axis_name="s")

    @functools.partial(
        pl.kernel,
        mesh=mesh,
        out_type=jax.ShapeDtypeStruct((n, d), jnp.float32),
        scratch_types=[
            pltpu.VMEM((c_chunks, k_chunk), jnp.int32),
            pltpu.VMEM((b_per_w, d), jnp.float32),
            pltpu.SemaphoreType.DMA,
        ],
    )
    def gather_kernel(table_hbm, idx_hbm, out_hbm, idx_v, rows_v, sem):
        wid = lax.axis_index("s") * nc + lax.axis_index("c")
        pltpu.sync_copy(idx_hbm.at[wid], idx_v)
        copies = [
            pltpu.async_copy(
                table_hbm.at[idx_v.at[j]],
                rows_v.at[pl.ds(j * k_chunk, k_chunk)],
                sem,
            )
            for j in range(c_chunks)
        ]
        for cp in copies:
            cp.wait()
        pltpu.sync_copy(rows_v, out_hbm.at[pl.ds(wid * b_per_w, b_per_w)])

    return gather_kernel(table, idx3)


def _gru_last(bsz, nsteps, h_act, ac, bac, wc, uc, bc):
    hd = h_act.shape[1]
    zblk = jnp.zeros((bsz, hd), h_act.dtype)
    for _ in range(nsteps):
        tr = jnp.dot(h_act, ac, preferred_element_type=jnp.float32) + bac
        hs = h_act[bsz:]
        m = hs.shape[0]
        fwd = tr[:m, :hd]
        if m > bsz:
            bwd = jnp.concatenate([tr[2 * bsz :, hd : 2 * hd], zblk], axis=0)
        else:
            bwd = zblk
        agg = tr[bsz:, 2 * hd :] + fwd + bwd
        gw = jnp.dot(agg, wc, preferred_element_type=jnp.float32) + bc
        gu = jnp.dot(hs, uc, preferred_element_type=jnp.float32)
        r = jax.nn.sigmoid(gw[:, :hd] + gu[:, :hd])
        z = jax.nn.sigmoid(gw[:, hd : 2 * hd] + gu[:, hd : 2 * hd])
        nn = jnp.tanh(gw[:, 2 * hd :] + r * gu[:, 2 * hd :])
        h_act = (1.0 - z) * nn + z * hs
    return h_act


def _fused_body(bsz, nsteps, h_ref, ac_ref, bac_ref, wc_ref, uc_ref, bc_ref,
                wout_ref, bout_ref, out_ref, last_scr):
    @pl.when(pl.program_id(0) == 0)
    def _():
        last_scr[...] = _gru_last(
            bsz, nsteps, h_ref[...], ac_ref[...], bac_ref[...], wc_ref[...],
            uc_ref[...], bc_ref[...])

    wblk = wout_ref[...].astype(jnp.float32)
    out_ref[...] = (
        jnp.dot(last_scr[...], wblk, preferred_element_type=jnp.float32)
        + bout_ref[...]
    )


def _tc_fused(h0, ac, bac, wc, uc, bc, wout_bf16, bout, bsz, nsteps):
    n, hd = h0.shape
    _, vocab = wout_bf16.shape
    vb = 8192
    grid = (vocab + vb - 1) // vb
    return pl.pallas_call(
        functools.partial(_fused_body, bsz, nsteps),
        grid=(grid,),
        in_specs=[
            pl.BlockSpec((n, hd), lambda i: (0, 0)),
            pl.BlockSpec(ac.shape, lambda i: (0, 0)),
            pl.BlockSpec(bac.shape, lambda i: (0, 0)),
            pl.BlockSpec(wc.shape, lambda i: (0, 0)),
            pl.BlockSpec(uc.shape, lambda i: (0, 0)),
            pl.BlockSpec(bc.shape, lambda i: (0, 0)),
            pl.BlockSpec((hd, vb), lambda i: (0, i)),
            pl.BlockSpec((1, vb), lambda i: (0, i)),
        ],
        out_specs=pl.BlockSpec((bsz, vb), lambda i: (0, i)),
        out_shape=jax.ShapeDtypeStruct((bsz, vocab), jnp.float32),
        scratch_shapes=[pltpu.VMEM((bsz, hd), jnp.float32)],
    )(h0, ac, bac, wc, uc, bc, wout_bf16, bout.reshape(1, vocab))


def kernel(x, emb, A, bA, W, U, b, Wout, bout):
    bsz, seqlen = x.shape
    _, hd = emb.shape
    wn = _NUMSTEPS + 1
    assert seqlen >= wn
    xw = x[:, seqlen - wn :].T.reshape(-1).astype(jnp.int32)
    h0 = _sc_gather(emb, xw)
    ac = jnp.concatenate([A[1], A[2], A[3]], axis=1)
    bac = jnp.concatenate([bA[1], bA[2], bA[3]], axis=0).reshape(1, 3 * hd)
    wc = jnp.concatenate([W[0], W[1], W[2]], axis=1)
    uc = jnp.concatenate([U[0], U[1], U[2]], axis=1)
    bc = jnp.concatenate([b[0], b[1], b[2]], axis=0).reshape(1, 3 * hd)
    return _tc_fused(h0, ac, bac, wc, uc, bc, Wout.astype(jnp.bfloat16),
                     bout, bsz, _NUMSTEPS)


# speedup vs baseline: 1.0167x; 1.0167x over previous
import functools

import jax
import jax.numpy as jnp
from jax import lax
from jax.experimental import pallas as pl
from jax.experimental.pallas import tpu as pltpu
from jax.experimental.pallas import tpu_sc as plsc

_NUMSTEPS = 10


def _chunk_plan(b_per_w):
    for c in range(1, b_per_w + 1):
        if b_per_w % c == 0:
            k = b_per_w // c
            if k <= 128 and k % 8 == 0:
                return c, k
    raise ValueError(f"cannot chunk {b_per_w} rows per worker")


def _sc_gather(table, idx):
    v, d = table.shape
    (n,) = idx.shape
    info = plsc.get_sparse_core_info()
    nc, ns = info.num_cores, info.num_subcores
    nw = nc * ns
    assert n % nw == 0
    b_per_w = n // nw
    c_chunks, k_chunk = _chunk_plan(b_per_w)
    idx3 = idx.reshape(nw, c_chunks, k_chunk)
    mesh = plsc.VectorSubcoreMesh(core_axis_name="c", subcore_axis_name="s")

    @functools.partial(
        pl.kernel,
        mesh=mesh,
        out_type=jax.ShapeDtypeStruct((n, d), jnp.float32),
        scratch_types=[
            pltpu.VMEM((c_chunks, k_chunk), jnp.int32),
            pltpu.VMEM((b_per_w, d), jnp.float32),
            pltpu.SemaphoreType.DMA,
        ],
    )
    def gather_kernel(table_hbm, idx_hbm, out_hbm, idx_v, rows_v, sem):
        wid = lax.axis_index("s") * nc + lax.axis_index("c")
        pltpu.sync_copy(idx_hbm.at[wid], idx_v)
        copies = [
            pltpu.async_copy(
                table_hbm.at[idx_v.at[j]],
                rows_v.at[pl.ds(j * k_chunk, k_chunk)],
                sem,
            )
            for j in range(c_chunks)
        ]
        for cp in copies:
            cp.wait()
        pltpu.sync_copy(rows_v, out_hbm.at[pl.ds(wid * b_per_w, b_per_w)])

    return gather_kernel(table, idx3)


def _gru_last(bsz, nsteps, h_act, ac, bac, wc, uc, bc):
    hd = h_act.shape[1]
    zblk = jnp.zeros((bsz, hd), h_act.dtype)
    for _ in range(nsteps):
        tr = jnp.dot(h_act, ac, preferred_element_type=jnp.float32) + bac
        hs = h_act[bsz:]
        m = hs.shape[0]
        fwd = tr[:m, :hd]
        if m > bsz:
            bwd = jnp.concatenate([tr[2 * bsz :, hd : 2 * hd], zblk], axis=0)
        else:
            bwd = zblk
        agg = tr[bsz:, 2 * hd :] + fwd + bwd
        gw = jnp.dot(agg, wc, preferred_element_type=jnp.float32) + bc
        gu = jnp.dot(hs, uc, preferred_element_type=jnp.float32)
        r = jax.nn.sigmoid(gw[:, :hd] + gu[:, :hd])
        z = jax.nn.sigmoid(gw[:, hd : 2 * hd] + gu[:, hd : 2 * hd])
        nn = jnp.tanh(gw[:, 2 * hd :] + r * gu[:, 2 * hd :])
        h_act = (1.0 - z) * nn + z * hs
    return h_act


def _fused_body(bsz, nsteps, h_ref, ac_ref, bac_ref, wc_ref, uc_ref, bc_ref,
                wout_ref, bout_ref, out_ref, last_scr):
    @pl.when(pl.program_id(0) == 0)
    def _():
        last_scr[...] = _gru_last(
            bsz, nsteps, h_ref[...], ac_ref[...], bac_ref[...], wc_ref[...],
            uc_ref[...], bc_ref[...])

    out_ref[...] = (
        jnp.dot(last_scr[...], wout_ref[...],
                preferred_element_type=jnp.float32)
        + bout_ref[...]
    )


def _tc_fused(h0, ac, bac, wc, uc, bc, wout, bout, bsz, nsteps):
    n, hd = h0.shape
    _, vocab = wout.shape
    vb = 8192
    grid = (vocab + vb - 1) // vb
    return pl.pallas_call(
        functools.partial(_fused_body, bsz, nsteps),
        grid=(grid,),
        in_specs=[
            pl.BlockSpec((n, hd), lambda i: (0, 0)),
            pl.BlockSpec(ac.shape, lambda i: (0, 0)),
            pl.BlockSpec(bac.shape, lambda i: (0, 0)),
            pl.BlockSpec(wc.shape, lambda i: (0, 0)),
            pl.BlockSpec(uc.shape, lambda i: (0, 0)),
            pl.BlockSpec(bc.shape, lambda i: (0, 0)),
            pl.BlockSpec((hd, vb), lambda i: (0, i)),
            pl.BlockSpec((1, vb), lambda i: (0, i)),
        ],
        out_specs=pl.BlockSpec((bsz, vb), lambda i: (0, i)),
        out_shape=jax.ShapeDtypeStruct((bsz, vocab), jnp.float32),
        scratch_shapes=[pltpu.VMEM((bsz, hd), jnp.float32)],
    )(h0, ac, bac, wc, uc, bc, wout, bout.reshape(1, vocab))


def kernel(x, emb, A, bA, W, U, b, Wout, bout):
    bsz, seqlen = x.shape
    _, hd = emb.shape
    wn = _NUMSTEPS + 1
    assert seqlen >= wn
    xw = x[:, seqlen - wn :].T.reshape(-1).astype(jnp.int32)
    h0 = _sc_gather(emb, xw)
    ac = jnp.concatenate([A[1], A[2], A[3]], axis=1)
    bac = jnp.concatenate([bA[1], bA[2], bA[3]], axis=0).reshape(1, 3 * hd)
    wc = jnp.concatenate([W[0], W[1], W[2]], axis=1)
    uc = jnp.concatenate([U[0], U[1], U[2]], axis=1)
    bc = jnp.concatenate([b[0], b[1], b[2]], axis=0).reshape(1, 3 * hd)
    return _tc_fused(h0, ac, bac, wc, uc, bc, Wout, bout, bsz, _NUMSTEPS)
